# baseline (speedup 1.0000x reference)
import functools

import jax
import jax.numpy as jnp
from jax import lax
from jax.experimental import pallas as pl
from jax.experimental.pallas import tpu as pltpu
from jax.experimental.pallas import tpu_sc as plsc

VOCAB = 1000
VSLABS = VOCAB // 8
CPAD = 1024
N = 51200
NPAD = N + 256
NTCOL = N // 128
NW = 32
WMAX = 13 * 128
LANES = 16
SG = 64


def _lse_body(t_ref, o_ref):
    x = t_ref[...]
    m = jnp.max(x, axis=1, keepdims=True)
    s = jnp.sum(jnp.exp(x - m), axis=1, keepdims=True)
    o_ref[...] = m + jnp.log(s)


def _fin_body(p_ref, o_ref):
    o_ref[...] = jnp.reshape(jnp.sum(p_ref[...]) * (1.0 / N), (1, 1))


def _sc_gather(idx_hbm, tgt_hbm, ttf_hbm, tblf_hbm, lse_hbm,
               out_hbm, psum_hbm,
               idx_v, tgt_v, fidx_v, tv_v, lsev_v,
               slab0, slab1, st0, st1, acc_v,
               ssem, lsem0, lsem1, osem0, osem1):
    wid = lax.axis_index("s") * 2 + lax.axis_index("c")
    nbc = jnp.where(wid < 16, 13, 12)
    start = wid * 12 + jnp.minimum(wid, 16)
    i0 = start * 128

    pltpu.sync_copy(idx_hbm.at[pl.ds(i0, WMAX)], idx_v)
    pltpu.sync_copy(tgt_hbm.at[pl.ds(i0, WMAX)], tgt_v)

    def flat_ids(g, _):
        sl = pl.ds(g * LANES, LANES)
        fidx_v[sl] = idx_v[sl] * VOCAB + tgt_v[sl]
        return 0
    lax.fori_loop(0, nbc * 8, flat_ids, 0)

    def elem_gather(k, _):
        sl = pl.ds(k * SG, SG)
        cp = pltpu.make_async_copy(tblf_hbm.at[fidx_v.at[sl]], tv_v.at[sl],
                                   ssem)
        cp.start()
        cp.wait()
        cp = pltpu.make_async_copy(lse_hbm.at[idx_v.at[sl]], lsev_v.at[sl],
                                   ssem)
        cp.start()
        cp.wait()
        return 0
    lax.fori_loop(0, nbc * 2, elem_gather, 0)

    def loss_acc(g, acc):
        sl = pl.ds(g * LANES, LANES)
        return acc + (lsev_v[sl] - tv_v[sl])
    acc = lax.fori_loop(0, nbc * 8, loss_acc,
                        jnp.zeros((LANES,), jnp.float32))
    acc_v[...] = acc
    pltpu.sync_copy(acc_v, psum_hbm.at[pl.ds(wid * LANES, LANES)])

    def slab_start(vt, slab, lsem):
        pltpu.make_async_copy(
            ttf_hbm.at[pl.ds(vt * (8 * CPAD), 8 * CPAD)], slab, lsem).start()

    def slab_wait(vt, slab, lsem):
        pltpu.make_async_copy(
            ttf_hbm.at[pl.ds(vt * (8 * CPAD), 8 * CPAD)], slab, lsem).wait()

    def tile_out_start(vt, t, st, osem):
        pltpu.make_async_copy(
            st, out_hbm.at[pl.ds(vt * 8, 8), pl.ds((start + t) * 128, 128)],
            osem).start()

    def tile_out_wait(st, osem):
        pltpu.make_async_copy(
            st, out_hbm.at[pl.ds(0, 8), pl.ds(0, 128)], osem).wait()

    def tile_compute(slab, t, st):
        for g in range(8):
            idxg = idx_v[pl.ds(t * 128 + g * LANES, LANES)]
            for j in range(8):
                v = plsc.load_gather(slab, [idxg + (j * CPAD)])
                st[j, pl.ds(g * LANES, LANES)] = v

    slab_start(0, slab0, lsem0)

    def vt_body(vt, k):
        @pl.when(lax.rem(vt, 2) == 0)
        def _():
            slab_wait(vt, slab0, lsem0)
        @pl.when(lax.rem(vt, 2) == 1)
        def _():
            slab_wait(vt, slab1, lsem1)

        @pl.when(vt + 1 < VSLABS)
        def _():
            @pl.when(lax.rem(vt, 2) == 0)
            def _():
                slab_start(vt + 1, slab1, lsem1)
            @pl.when(lax.rem(vt, 2) == 1)
            def _():
                slab_start(vt + 1, slab0, lsem0)

        def tile_body(t, k):
            kk = k + t

            def work(slab, st, osem):
                @pl.when(kk >= 2)
                def _():
                    tile_out_wait(st, osem)
                tile_compute(slab, t, st)
                tile_out_start(vt, t, st, osem)

            @pl.when((lax.rem(kk, 2) == 0) & (lax.rem(vt, 2) == 0))
            def _():
                work(slab0, st0, osem0)
            @pl.when((lax.rem(kk, 2) == 1) & (lax.rem(vt, 2) == 0))
            def _():
                work(slab0, st1, osem1)
            @pl.when((lax.rem(kk, 2) == 0) & (lax.rem(vt, 2) == 1))
            def _():
                work(slab1, st0, osem0)
            @pl.when((lax.rem(kk, 2) == 1) & (lax.rem(vt, 2) == 1))
            def _():
                work(slab1, st1, osem1)
            return k

        lax.fori_loop(0, nbc, lambda t, kk_: tile_body(t, k), 0)
        return k + nbc

    lax.fori_loop(0, VSLABS, vt_body, 0)
    tile_out_wait(st0, osem0)
    tile_out_wait(st1, osem1)


def kernel(idx, targets, table):
    idx_flat = idx.reshape(-1).astype(jnp.int32)
    tgt_flat = targets.reshape(-1).astype(jnp.int32)
    idxp = jnp.pad(idx_flat, (0, NPAD - N))
    tgtp = jnp.pad(tgt_flat, (0, NPAD - N))
    ttf = jnp.pad(table.T, ((0, 0), (0, CPAD - VOCAB))).reshape(-1)
    tblf = table.reshape(-1)

    lse = pl.pallas_call(
        _lse_body,
        out_shape=jax.ShapeDtypeStruct((VOCAB, 1), jnp.float32),
    )(table)
    lse = lse.reshape(VOCAB)

    sc = functools.partial(
        pl.kernel,
        mesh=plsc.VectorSubcoreMesh(core_axis_name="c", subcore_axis_name="s"),
        out_type=[
            jax.ShapeDtypeStruct((VOCAB, N), jnp.float32),
            jax.ShapeDtypeStruct((NW * LANES,), jnp.float32),
        ],
        scratch_types=[
            pltpu.VMEM((WMAX,), jnp.int32),
            pltpu.VMEM((WMAX,), jnp.int32),
            pltpu.VMEM((WMAX,), jnp.int32),
            pltpu.VMEM((WMAX,), jnp.float32),
            pltpu.VMEM((WMAX,), jnp.float32),
            pltpu.VMEM((8 * CPAD,), jnp.float32),
            pltpu.VMEM((8 * CPAD,), jnp.float32),
            pltpu.VMEM((8, 128), jnp.float32),
            pltpu.VMEM((8, 128), jnp.float32),
            pltpu.VMEM((LANES,), jnp.float32),
            pltpu.SemaphoreType.DMA,
            pltpu.SemaphoreType.DMA,
            pltpu.SemaphoreType.DMA,
            pltpu.SemaphoreType.DMA,
            pltpu.SemaphoreType.DMA,
        ],
        compiler_params=pltpu.CompilerParams(
            use_tc_tiling_on_sc=True, needs_layout_passes=False),
    )(_sc_gather)
    outT, psums = sc(idxp, tgtp, ttf, tblf, lse)
    logits2 = outT.T

    fin = pl.pallas_call(
        _fin_body,
        out_shape=jax.ShapeDtypeStruct((1, 1), jnp.float32),
    )(psums)
    loss = fin[0, 0]
    return (logits2, loss)


# speedup vs baseline: 1.0004x; 1.0004x over previous
import functools

import jax
import jax.numpy as jnp
from jax import lax
from jax.experimental import pallas as pl
from jax.experimental.pallas import tpu as pltpu
from jax.experimental.pallas import tpu_sc as plsc

VOCAB = 1000
VSLABS = VOCAB // 8
CPAD = 1024
N = 51200
NPAD = N + 256
NTCOL = N // 128
NW = 32
WMAX = 13 * 128
LANES = 16
SG = 64


def _lse_body(t_ref, o_ref):
    x = t_ref[...]
    m = jnp.max(x, axis=1, keepdims=True)
    s = jnp.sum(jnp.exp(x - m), axis=1, keepdims=True)
    o_ref[...] = m + jnp.log(s)


def _fin_body(p_ref, o_ref):
    o_ref[...] = jnp.reshape(jnp.sum(p_ref[...]) * (1.0 / N), (1, 1))


def _sc_gather(idx_hbm, tgt_hbm, ttf_hbm, tblf_hbm, lse_hbm,
               out_hbm, psum_hbm,
               idx_v, tgt_v, fidx_v, tv_v, lsev_v,
               slab0, slab1, st0, st1, acc_v,
               ssem, lsem0, lsem1, osem0, osem1):
    wid = lax.axis_index("s") * 2 + lax.axis_index("c")
    nbc = jnp.where(wid < 16, 13, 12)
    start = wid * 12 + jnp.minimum(wid, 16)
    i0 = start * 128

    pltpu.sync_copy(idx_hbm.at[pl.ds(i0, WMAX)], idx_v)
    pltpu.sync_copy(tgt_hbm.at[pl.ds(i0, WMAX)], tgt_v)

    def flat_ids(g, _):
        sl = pl.ds(g * LANES, LANES)
        fidx_v[sl] = idx_v[sl] * VOCAB + tgt_v[sl]
        return 0
    lax.fori_loop(0, nbc * 8, flat_ids, 0)

    def elem_gather(k, _):
        sl = pl.ds(k * SG, SG)
        cp = pltpu.make_async_copy(tblf_hbm.at[fidx_v.at[sl]], tv_v.at[sl],
                                   ssem)
        cp.start()
        cp.wait()
        cp = pltpu.make_async_copy(lse_hbm.at[idx_v.at[sl]], lsev_v.at[sl],
                                   ssem)
        cp.start()
        cp.wait()
        return 0
    lax.fori_loop(0, nbc * 2, elem_gather, 0)

    def loss_acc(g, acc):
        sl = pl.ds(g * LANES, LANES)
        return acc + (lsev_v[sl] - tv_v[sl])
    acc = lax.fori_loop(0, nbc * 8, loss_acc,
                        jnp.zeros((LANES,), jnp.float32))
    acc_v[...] = acc
    pltpu.sync_copy(acc_v, psum_hbm.at[pl.ds(wid * LANES, LANES)])

    def slab_start(vt, slab, lsem):
        pltpu.make_async_copy(
            ttf_hbm.at[pl.ds(vt * (8 * CPAD), 8 * CPAD)], slab, lsem).start()

    def slab_wait(vt, slab, lsem):
        pltpu.make_async_copy(
            ttf_hbm.at[pl.ds(vt * (8 * CPAD), 8 * CPAD)], slab, lsem).wait()

    def tile_out_start(vt, t, st, osem):
        pltpu.make_async_copy(
            st, out_hbm.at[pl.ds(vt * 8, 8), pl.ds((start + t) * 128, 128)],
            osem).start()

    def tile_out_wait(st, osem):
        pltpu.make_async_copy(
            st, out_hbm.at[pl.ds(0, 8), pl.ds(0, 128)], osem).wait()

    def tile_compute(slab, t, st):
        for g in range(8):
            idxg = idx_v[pl.ds(t * 128 + g * LANES, LANES)]
            for j in range(8):
                v = plsc.load_gather(slab, [idxg + (j * CPAD)])
                st[j, pl.ds(g * LANES, LANES)] = v

    slab_start(0, slab0, lsem0)

    def vt_body(vt, k):
        @pl.when(lax.rem(vt, 2) == 0)
        def _():
            slab_wait(vt, slab0, lsem0)
        @pl.when(lax.rem(vt, 2) == 1)
        def _():
            slab_wait(vt, slab1, lsem1)

        @pl.when(vt + 1 < VSLABS)
        def _():
            @pl.when(lax.rem(vt, 2) == 0)
            def _():
                slab_start(vt + 1, slab1, lsem1)
            @pl.when(lax.rem(vt, 2) == 1)
            def _():
                slab_start(vt + 1, slab0, lsem0)

        def tile_body(t, k):
            kk = k + t

            def work(slab, st, osem):
                @pl.when(kk >= 2)
                def _():
                    tile_out_wait(st, osem)
                tile_compute(slab, t, st)
                tile_out_start(vt, t, st, osem)

            @pl.when((lax.rem(kk, 2) == 0) & (lax.rem(vt, 2) == 0))
            def _():
                work(slab0, st0, osem0)
            @pl.when((lax.rem(kk, 2) == 1) & (lax.rem(vt, 2) == 0))
            def _():
                work(slab0, st1, osem1)
            @pl.when((lax.rem(kk, 2) == 0) & (lax.rem(vt, 2) == 1))
            def _():
                work(slab1, st0, osem0)
            @pl.when((lax.rem(kk, 2) == 1) & (lax.rem(vt, 2) == 1))
            def _():
                work(slab1, st1, osem1)
            return k

        lax.fori_loop(0, nbc, lambda t, kk_: tile_body(t, k), 0)
        return k + nbc

    lax.fori_loop(0, VSLABS, vt_body, 0)
    tile_out_wait(st0, osem0)
    tile_out_wait(st1, osem1)


def kernel(idx, targets, table):
    idx_flat = idx.reshape(-1).astype(jnp.int32)
    tgt_flat = targets.reshape(-1).astype(jnp.int32)
    idxp = jnp.pad(idx_flat, (0, NPAD - N))
    tgtp = jnp.pad(tgt_flat, (0, NPAD - N))
    ttf = jnp.pad(table.T, ((0, 0), (0, CPAD - VOCAB))).reshape(-1)
    tblf = table.reshape(-1)

    lse = pl.pallas_call(
        _lse_body,
        out_shape=jax.ShapeDtypeStruct((VOCAB, 1), jnp.float32),
    )(table)
    lse = lse.reshape(VOCAB)

    sc = functools.partial(
        pl.kernel,
        mesh=plsc.VectorSubcoreMesh(core_axis_name="c", subcore_axis_name="s"),
        out_type=[
            jax.ShapeDtypeStruct((VOCAB, N), jnp.float32),
            jax.ShapeDtypeStruct((NW * LANES,), jnp.float32),
        ],
        scratch_types=[
            pltpu.VMEM((WMAX,), jnp.int32),
            pltpu.VMEM((WMAX,), jnp.int32),
            pltpu.VMEM((WMAX,), jnp.int32),
            pltpu.VMEM((WMAX,), jnp.float32),
            pltpu.VMEM((WMAX,), jnp.float32),
            pltpu.VMEM((8 * CPAD,), jnp.float32),
            pltpu.VMEM((8 * CPAD,), jnp.float32),
            pltpu.VMEM((8, 128), jnp.float32),
            pltpu.VMEM((8, 128), jnp.float32),
            pltpu.VMEM((LANES,), jnp.float32),
            pltpu.SemaphoreType.DMA,
            pltpu.SemaphoreType.DMA,
            pltpu.SemaphoreType.DMA,
            pltpu.SemaphoreType.DMA,
            pltpu.SemaphoreType.DMA,
        ],
        compiler_params=pltpu.CompilerParams(
            use_tc_tiling_on_sc=True, needs_layout_passes=False,
            disable_bounds_checks=True),
    )(_sc_gather)
    outT, psums = sc(idxp, tgtp, ttf, tblf, lse)
    logits2 = outT.T

    fin = pl.pallas_call(
        _fin_body,
        out_shape=jax.ShapeDtypeStruct((1, 1), jnp.float32),
    )(psums)
    loss = fin[0, 0]
    return (logits2, loss)


# speedup vs baseline: 2.3232x; 2.3222x over previous
import functools

import jax
import jax.numpy as jnp
from jax import lax
from jax.experimental import pallas as pl
from jax.experimental.pallas import tpu as pltpu
from jax.experimental.pallas import tpu_sc as plsc

VOCAB = 1000
VSLABS = VOCAB // 8
CPAD = 1024
N = 51200
NW = 32
WMAX = 14 * 128
LANES = 16
SG = 64


def _lse_body(t_ref, o_ref):
    x = t_ref[...]
    m = jnp.max(x, axis=1, keepdims=True)
    s = jnp.sum(jnp.exp(x - m), axis=1, keepdims=True)
    o_ref[...] = m + jnp.log(s)


def _fin_body(p_ref, o_ref):
    o_ref[...] = jnp.reshape(jnp.sum(p_ref[...]) * (1.0 / N), (1, 1))


def _sc_gather(idx_hbm, tgt_hbm, ttf_hbm, tblf_hbm, lse_hbm,
               out_hbm, psum_hbm,
               idx_v, tgt_v, fidx_v, tv_v, lsev_v,
               slab0, slab1, st0, st1, acc_v,
               ssem, lsem0, lsem1, osem0, osem1):
    wid = lax.axis_index("s") * 2 + lax.axis_index("c")

    def slab_start(vt, slab, lsem):
        pltpu.make_async_copy(
            ttf_hbm.at[pl.ds(vt * (8 * CPAD), 8 * CPAD)], slab, lsem).start()

    def slab_wait(vt, slab, lsem):
        pltpu.make_async_copy(
            ttf_hbm.at[pl.ds(vt * (8 * CPAD), 8 * CPAD)], slab, lsem).wait()

    def run(nbc, start):
        w = nbc * 128
        i0 = start * 128

        pltpu.sync_copy(idx_hbm.at[pl.ds(i0, w)], idx_v.at[pl.ds(0, w)])
        pltpu.sync_copy(tgt_hbm.at[pl.ds(i0, w)], tgt_v.at[pl.ds(0, w)])

        def flat_ids(g, _):
            sl = pl.ds(g * LANES, LANES)
            fidx_v[sl] = idx_v[sl] * VOCAB + tgt_v[sl]
            return 0
        lax.fori_loop(0, nbc * 8, flat_ids, 0)

        def elem_gather(k, _):
            sl = pl.ds(k * SG, SG)
            cp = pltpu.make_async_copy(tblf_hbm.at[fidx_v.at[sl]],
                                       tv_v.at[sl], ssem)
            cp.start()
            cp.wait()
            cp = pltpu.make_async_copy(lse_hbm.at[idx_v.at[sl]],
                                       lsev_v.at[sl], ssem)
            cp.start()
            cp.wait()
            return 0
        lax.fori_loop(0, nbc * 2, elem_gather, 0)

        def loss_acc(g, acc):
            sl = pl.ds(g * LANES, LANES)
            return acc + (lsev_v[sl] - tv_v[sl])
        acc = lax.fori_loop(0, nbc * 8, loss_acc,
                            jnp.zeros((LANES,), jnp.float32))
        acc_v[...] = acc
        pltpu.sync_copy(acc_v, psum_hbm.at[pl.ds(wid * LANES, LANES)])

        def stage_out_start(vt, st, osem):
            pltpu.make_async_copy(
                st.at[pl.ds(0, 8), pl.ds(0, w)],
                out_hbm.at[pl.ds(vt * 8, 8), pl.ds(i0, w)], osem).start()

        def stage_out_wait(st, osem):
            pltpu.make_async_copy(
                st.at[pl.ds(0, 8), pl.ds(0, w)],
                out_hbm.at[pl.ds(0, 8), pl.ds(i0, w)], osem).wait()

        def inner(vt, slab_a, lsem_a, slab_b, lsem_b, st, osem):
            slab_wait(vt, slab_a, lsem_a)

            @pl.when(vt + 1 < VSLABS)
            def _():
                slab_start(vt + 1, slab_b, lsem_b)

            @pl.when(vt >= 2)
            def _():
                stage_out_wait(st, osem)

            @plsc.parallel_loop(0, nbc * 8, unroll=2)
            def _(g):
                idxg = idx_v[pl.ds(g * LANES, LANES)]
                for j in range(8):
                    v = plsc.load_gather(slab_a, [idxg + (j * CPAD)])
                    st[j, pl.ds(g * LANES, LANES)] = v

            stage_out_start(vt, st, osem)

        slab_start(0, slab0, lsem0)

        def vt_body(vt, _):
            @pl.when(lax.rem(vt, 2) == 0)
            def _():
                inner(vt, slab0, lsem0, slab1, lsem1, st0, osem0)
            @pl.when(lax.rem(vt, 2) == 1)
            def _():
                inner(vt, slab1, lsem1, slab0, lsem0, st1, osem1)
            return 0

        lax.fori_loop(0, VSLABS, vt_body, 0)
        stage_out_wait(st0, osem0)
        stage_out_wait(st1, osem1)

    @pl.when(wid < 8)
    def _():
        run(14, wid * 14)

    @pl.when(wid >= 8)
    def _():
        run(12, 112 + (wid - 8) * 12)


def kernel(idx, targets, table):
    idx_flat = idx.reshape(-1).astype(jnp.int32)
    tgt_flat = targets.reshape(-1).astype(jnp.int32)
    ttf = jnp.pad(table.T, ((0, 0), (0, CPAD - VOCAB))).reshape(-1)
    tblf = table.reshape(-1)

    lse = pl.pallas_call(
        _lse_body,
        out_shape=jax.ShapeDtypeStruct((VOCAB, 1), jnp.float32),
    )(table)
    lse = lse.reshape(VOCAB)

    sc = functools.partial(
        pl.kernel,
        mesh=plsc.VectorSubcoreMesh(core_axis_name="c", subcore_axis_name="s"),
        out_type=[
            jax.ShapeDtypeStruct((VOCAB, N), jnp.float32),
            jax.ShapeDtypeStruct((NW * LANES,), jnp.float32),
        ],
        scratch_types=[
            pltpu.VMEM((WMAX,), jnp.int32),
            pltpu.VMEM((WMAX,), jnp.int32),
            pltpu.VMEM((WMAX,), jnp.int32),
            pltpu.VMEM((WMAX,), jnp.float32),
            pltpu.VMEM((WMAX,), jnp.float32),
            pltpu.VMEM((8 * CPAD,), jnp.float32),
            pltpu.VMEM((8 * CPAD,), jnp.float32),
            pltpu.VMEM((8, WMAX), jnp.float32),
            pltpu.VMEM((8, WMAX), jnp.float32),
            pltpu.VMEM((LANES,), jnp.float32),
            pltpu.SemaphoreType.DMA,
            pltpu.SemaphoreType.DMA,
            pltpu.SemaphoreType.DMA,
            pltpu.SemaphoreType.DMA,
            pltpu.SemaphoreType.DMA,
        ],
        compiler_params=pltpu.CompilerParams(
            use_tc_tiling_on_sc=True, needs_layout_passes=False,
            disable_bounds_checks=True),
    )(_sc_gather)
    outT, psums = sc(idx_flat, tgt_flat, ttf, tblf, lse)
    logits2 = outT.T

    fin = pl.pallas_call(
        _fin_body,
        out_shape=jax.ShapeDtypeStruct((1, 1), jnp.float32),
    )(psums)
    loss = fin[0, 0]
    return (logits2, loss)


# speedup vs baseline: 3.1263x; 1.3457x over previous
import functools

import jax
import jax.numpy as jnp
from jax import lax
from jax.experimental import pallas as pl
from jax.experimental.pallas import tpu as pltpu
from jax.experimental.pallas import tpu_sc as plsc

VOCAB = 1000
VSLABS = VOCAB // 8
VHALF = 63
CPAD = 1024
N = 51200
NW = 32
WCOL = 25 * 128
PERW = N // NW
LANES = 16
SG = 64


def _lse_body(t_ref, o_ref):
    x = t_ref[...]
    m = jnp.max(x, axis=1, keepdims=True)
    s = jnp.sum(jnp.exp(x - m), axis=1, keepdims=True)
    o_ref[...] = m + jnp.log(s)


def _fin_body(p_ref, o_ref):
    o_ref[...] = jnp.reshape(jnp.sum(p_ref[...]) * (1.0 / N), (1, 1))


def _sc_gather(idx_hbm, tgt_hbm, ttf_hbm, tblf_hbm, lse_hbm,
               out_hbm, psum_hbm,
               idx_v, lidx_v, ltgt_v, fidx_v, tv_v, lsev_v,
               slab0, slab1, st0, st1, acc_v,
               ssem, lsem0, lsem1, osem0, osem1):
    wid = lax.axis_index("s") * 2 + lax.axis_index("c")
    pair = lax.div(wid, 2)
    half = lax.rem(wid, 2)
    i0 = pair * WCOL
    vt0 = half * VHALF
    nvt = jnp.where(half == 0, VHALF, VSLABS - VHALF)

    pltpu.sync_copy(idx_hbm.at[pl.ds(i0, WCOL)], idx_v)

    li0 = wid * PERW
    pltpu.sync_copy(idx_hbm.at[pl.ds(li0, PERW)], lidx_v)
    pltpu.sync_copy(tgt_hbm.at[pl.ds(li0, PERW)], ltgt_v)

    def flat_ids(g, _):
        sl = pl.ds(g * LANES, LANES)
        fidx_v[sl] = lidx_v[sl] * VOCAB + ltgt_v[sl]
        return 0
    lax.fori_loop(0, PERW // LANES, flat_ids, 0)

    def elem_gather(k, _):
        sl = pl.ds(k * SG, SG)
        cp = pltpu.make_async_copy(tblf_hbm.at[fidx_v.at[sl]],
                                   tv_v.at[sl], ssem)
        cp.start()
        cp.wait()
        cp = pltpu.make_async_copy(lse_hbm.at[lidx_v.at[sl]],
                                   lsev_v.at[sl], ssem)
        cp.start()
        cp.wait()
        return 0
    lax.fori_loop(0, PERW // SG, elem_gather, 0)

    def loss_acc(g, acc):
        sl = pl.ds(g * LANES, LANES)
        return acc + (lsev_v[sl] - tv_v[sl])
    acc = lax.fori_loop(0, PERW // LANES, loss_acc,
                        jnp.zeros((LANES,), jnp.float32))
    acc_v[...] = acc
    pltpu.sync_copy(acc_v, psum_hbm.at[pl.ds(wid * LANES, LANES)])

    def slab_start(vt, slab, lsem):
        pltpu.make_async_copy(
            ttf_hbm.at[pl.ds(vt * (8 * CPAD), 8 * CPAD)], slab, lsem).start()

    def slab_wait(vt, slab, lsem):
        pltpu.make_async_copy(
            ttf_hbm.at[pl.ds(vt * (8 * CPAD), 8 * CPAD)], slab, lsem).wait()

    def stage_out_start(vt, st, osem):
        pltpu.make_async_copy(
            st, out_hbm.at[pl.ds(vt * 8, 8), pl.ds(i0, WCOL)], osem).start()

    def stage_out_wait(st, osem):
        pltpu.make_async_copy(
            st, out_hbm.at[pl.ds(0, 8), pl.ds(i0, WCOL)], osem).wait()

    def inner(vt, k, slab_a, lsem_a, slab_b, lsem_b, st, osem):
        slab_wait(vt, slab_a, lsem_a)

        @pl.when(k + 1 < nvt)
        def _():
            slab_start(vt + 1, slab_b, lsem_b)

        @pl.when(k >= 2)
        def _():
            stage_out_wait(st, osem)

        @plsc.parallel_loop(0, WCOL // LANES, unroll=2)
        def _(g):
            idxg = idx_v[pl.ds(g * LANES, LANES)]
            for j in range(8):
                v = plsc.load_gather(slab_a, [idxg + (j * CPAD)])
                st[j, pl.ds(g * LANES, LANES)] = v

        stage_out_start(vt, st, osem)

    slab_start(vt0, slab0, lsem0)

    def vt_body(k, _):
        vt = vt0 + k
        @pl.when(lax.rem(k, 2) == 0)
        def _():
            inner(vt, k, slab0, lsem0, slab1, lsem1, st0, osem0)
        @pl.when(lax.rem(k, 2) == 1)
        def _():
            inner(vt, k, slab1, lsem1, slab0, lsem0, st1, osem1)
        return 0

    lax.fori_loop(0, nvt, vt_body, 0)
    stage_out_wait(st0, osem0)
    stage_out_wait(st1, osem1)


def kernel(idx, targets, table):
    idx_flat = idx.reshape(-1).astype(jnp.int32)
    tgt_flat = targets.reshape(-1).astype(jnp.int32)
    ttf = jnp.pad(table.T, ((0, 0), (0, CPAD - VOCAB))).reshape(-1)
    tblf = table.reshape(-1)

    lse = pl.pallas_call(
        _lse_body,
        out_shape=jax.ShapeDtypeStruct((VOCAB, 1), jnp.float32),
    )(table)
    lse = lse.reshape(VOCAB)

    sc = functools.partial(
        pl.kernel,
        mesh=plsc.VectorSubcoreMesh(core_axis_name="c", subcore_axis_name="s"),
        out_type=[
            jax.ShapeDtypeStruct((VOCAB, N), jnp.float32),
            jax.ShapeDtypeStruct((NW * LANES,), jnp.float32),
        ],
        scratch_types=[
            pltpu.VMEM((WCOL,), jnp.int32),
            pltpu.VMEM((PERW,), jnp.int32),
            pltpu.VMEM((PERW,), jnp.int32),
            pltpu.VMEM((PERW,), jnp.int32),
            pltpu.VMEM((PERW,), jnp.float32),
            pltpu.VMEM((PERW,), jnp.float32),
            pltpu.VMEM((8 * CPAD,), jnp.float32),
            pltpu.VMEM((8 * CPAD,), jnp.float32),
            pltpu.VMEM((8, WCOL), jnp.float32),
            pltpu.VMEM((8, WCOL), jnp.float32),
            pltpu.VMEM((LANES,), jnp.float32),
            pltpu.SemaphoreType.DMA,
            pltpu.SemaphoreType.DMA,
            pltpu.SemaphoreType.DMA,
            pltpu.SemaphoreType.DMA,
            pltpu.SemaphoreType.DMA,
        ],
        compiler_params=pltpu.CompilerParams(
            use_tc_tiling_on_sc=True, needs_layout_passes=False,
            disable_bounds_checks=True),
    )(_sc_gather)
    outT, psums = sc(idx_flat, tgt_flat, ttf, tblf, lse)
    logits2 = outT.T

    fin = pl.pallas_call(
        _fin_body,
        out_shape=jax.ShapeDtypeStruct((1, 1), jnp.float32),
    )(psums)
    loss = fin[0, 0]
    return (logits2, loss)


# speedup vs baseline: 3.2663x; 1.0448x over previous
import functools

import jax
import jax.numpy as jnp
from jax import lax
from jax.experimental import pallas as pl
from jax.experimental.pallas import tpu as pltpu
from jax.experimental.pallas import tpu_sc as plsc

VOCAB = 1000
VSLABS = VOCAB // 8
VHALF = 63
CPAD = 1000
N = 51200
NW = 32
WCOL = 25 * 128
PERW = N // NW
LANES = 16
SG = 64


def _lse_body(t_ref, o_ref):
    x = t_ref[...]
    m = jnp.max(x, axis=1, keepdims=True)
    s = jnp.sum(jnp.exp(x - m), axis=1, keepdims=True)
    o_ref[...] = m + jnp.log(s)


def _fin_body(p_ref, o_ref):
    o_ref[...] = jnp.reshape(jnp.sum(p_ref[...]) * (1.0 / N), (1, 1))


def _sc_gather(idx_hbm, tgt_hbm, ttf_hbm, lse_hbm,
               out_hbm, psum_hbm,
               idx_v, lidx_v, ltgt_v, fidx_v, tv_v, lsev_v,
               slab0, slab1, st0, st1, acc_v,
               ssem, lsem0, lsem1, osem0a, osem0b, osem1a, osem1b):
    wid = lax.axis_index("s") * 2 + lax.axis_index("c")
    pair = lax.div(wid, 2)
    half = lax.rem(wid, 2)
    i0 = pair * WCOL
    vt0 = half * VHALF
    nvt = jnp.where(half == 0, VHALF, VSLABS - VHALF)

    pltpu.sync_copy(idx_hbm.at[pl.ds(i0, WCOL)], idx_v)

    li0 = wid * PERW
    pltpu.sync_copy(idx_hbm.at[pl.ds(li0, PERW)], lidx_v)
    pltpu.sync_copy(tgt_hbm.at[pl.ds(li0, PERW)], ltgt_v)

    def flat_ids(g, _):
        sl = pl.ds(g * LANES, LANES)
        fidx_v[sl] = ltgt_v[sl] * VOCAB + lidx_v[sl]
        return 0
    lax.fori_loop(0, PERW // LANES, flat_ids, 0)

    def elem_gather_start(k, _):
        sl = pl.ds(k * SG, SG)
        pltpu.make_async_copy(ttf_hbm.at[fidx_v.at[sl]],
                              tv_v.at[sl], ssem).start()
        pltpu.make_async_copy(lse_hbm.at[lidx_v.at[sl]],
                              lsev_v.at[sl], ssem).start()
        return 0
    lax.fori_loop(0, PERW // SG, elem_gather_start, 0)

    def loss_tail():
        def elem_gather_wait(k, _):
            sl = pl.ds(k * SG, SG)
            pltpu.make_async_copy(ttf_hbm.at[fidx_v.at[sl]],
                                  tv_v.at[sl], ssem).wait()
            pltpu.make_async_copy(lse_hbm.at[lidx_v.at[sl]],
                                  lsev_v.at[sl], ssem).wait()
            return 0
        lax.fori_loop(0, PERW // SG, elem_gather_wait, 0)

        def loss_acc(g, acc):
            sl = pl.ds(g * LANES, LANES)
            return acc + (lsev_v[sl] - tv_v[sl])
        acc = lax.fori_loop(0, PERW // LANES, loss_acc,
                            jnp.zeros((LANES,), jnp.float32))
        acc_v[...] = acc
        pltpu.sync_copy(acc_v, psum_hbm.at[pl.ds(wid * LANES, LANES)])

    def slab_start(vt, slab, lsem):
        pltpu.make_async_copy(
            ttf_hbm.at[pl.ds(vt * (8 * CPAD), 8 * CPAD)], slab, lsem).start()

    def slab_wait(vt, slab, lsem):
        pltpu.make_async_copy(
            ttf_hbm.at[pl.ds(vt * (8 * CPAD), 8 * CPAD)], slab, lsem).wait()

    HA = 1536
    HB = WCOL - HA

    def half_start(vt, st, osem, off, hw):
        pltpu.make_async_copy(
            st.at[pl.ds(0, 8), pl.ds(off, hw)],
            out_hbm.at[pl.ds(vt * 8, 8), pl.ds(i0 + off, hw)], osem).start()

    def half_wait(st, osem, off, hw):
        pltpu.make_async_copy(
            st.at[pl.ds(0, 8), pl.ds(off, hw)],
            out_hbm.at[pl.ds(0, 8), pl.ds(i0 + off, hw)], osem).wait()

    def fill(slab_a, st, g_lo, g_hi):
        @plsc.parallel_loop(g_lo, g_hi, unroll=4)
        def _(g):
            idxg = idx_v[pl.ds(g * LANES, LANES)]
            for j in range(8):
                v = plsc.load_gather(slab_a, [idxg + (j * CPAD)])
                st[j, pl.ds(g * LANES, LANES)] = v

    def inner(vt, k, slab_a, lsem_a, slab_b, lsem_b, st, osem_a, osem_b):
        slab_wait(vt, slab_a, lsem_a)

        @pl.when(k + 1 < nvt)
        def _():
            slab_start(vt + 1, slab_b, lsem_b)

        @pl.when(k >= 2)
        def _():
            half_wait(st, osem_a, 0, HA)
        fill(slab_a, st, 0, HA // LANES)
        half_start(vt, st, osem_a, 0, HA)

        @pl.when(k >= 2)
        def _():
            half_wait(st, osem_b, HA, HB)
        fill(slab_a, st, HA // LANES, WCOL // LANES)
        half_start(vt, st, osem_b, HA, HB)

    slab_start(vt0, slab0, lsem0)

    def vt_body(k, _):
        vt = vt0 + k
        @pl.when(lax.rem(k, 2) == 0)
        def _():
            inner(vt, k, slab0, lsem0, slab1, lsem1, st0, osem0a, osem0b)
        @pl.when(lax.rem(k, 2) == 1)
        def _():
            inner(vt, k, slab1, lsem1, slab0, lsem0, st1, osem1a, osem1b)
        return 0

    lax.fori_loop(0, nvt, vt_body, 0)
    loss_tail()
    half_wait(st0, osem0a, 0, HA)
    half_wait(st0, osem0b, HA, HB)
    half_wait(st1, osem1a, 0, HA)
    half_wait(st1, osem1b, HA, HB)


def kernel(idx, targets, table):
    idx_flat = idx.reshape(-1).astype(jnp.int32)
    tgt_flat = targets.reshape(-1).astype(jnp.int32)
    ttf = table.T.reshape(-1)

    lse = pl.pallas_call(
        _lse_body,
        out_shape=jax.ShapeDtypeStruct((VOCAB, 1), jnp.float32),
    )(table)
    lse = lse.reshape(VOCAB)

    sc = functools.partial(
        pl.kernel,
        mesh=plsc.VectorSubcoreMesh(core_axis_name="c", subcore_axis_name="s"),
        out_type=[
            jax.ShapeDtypeStruct((VOCAB, N), jnp.float32),
            jax.ShapeDtypeStruct((NW * LANES,), jnp.float32),
        ],
        scratch_types=[
            pltpu.VMEM((WCOL,), jnp.int32),
            pltpu.VMEM((PERW,), jnp.int32),
            pltpu.VMEM((PERW,), jnp.int32),
            pltpu.VMEM((PERW,), jnp.int32),
            pltpu.VMEM((PERW,), jnp.float32),
            pltpu.VMEM((PERW,), jnp.float32),
            pltpu.VMEM((8 * CPAD,), jnp.float32),
            pltpu.VMEM((8 * CPAD,), jnp.float32),
            pltpu.VMEM((8, WCOL), jnp.float32),
            pltpu.VMEM((8, WCOL), jnp.float32),
            pltpu.VMEM((LANES,), jnp.float32),
            pltpu.SemaphoreType.DMA,
            pltpu.SemaphoreType.DMA,
            pltpu.SemaphoreType.DMA,
            pltpu.SemaphoreType.DMA,
            pltpu.SemaphoreType.DMA,
            pltpu.SemaphoreType.DMA,
            pltpu.SemaphoreType.DMA,
        ],
        compiler_params=pltpu.CompilerParams(
            use_tc_tiling_on_sc=True, needs_layout_passes=False,
            disable_bounds_checks=True),
    )(_sc_gather)
    outT, psums = sc(idx_flat, tgt_flat, ttf, lse)
    logits2 = outT.T

    fin = pl.pallas_call(
        _fin_body,
        out_shape=jax.ShapeDtypeStruct((1, 1), jnp.float32),
    )(psums)
    loss = fin[0, 0]
    return (logits2, loss)


# speedup vs baseline: 3.2876x; 1.0065x over previous
import functools

import jax
import jax.numpy as jnp
from jax import lax
from jax.experimental import pallas as pl
from jax.experimental.pallas import tpu as pltpu
from jax.experimental.pallas import tpu_sc as plsc

VOCAB = 1000
VSLABS = VOCAB // 8
VHALF = 63
CPAD = 1000
N = 51200
NW = 32
WCOL = 25 * 128
PERW = N // NW
LANES = 16
SG = 64


def _lse_body(t_ref, o_ref):
    x = t_ref[...]
    m = jnp.max(x, axis=1, keepdims=True)
    s = jnp.sum(jnp.exp(x - m), axis=1, keepdims=True)
    o_ref[...] = m + jnp.log(s)


def _fin_body(p_ref, o_ref):
    o_ref[...] = jnp.reshape(jnp.sum(p_ref[...]) * (1.0 / N), (1, 1))


def _sc_gather(idx_hbm, tgt_hbm, ttf_hbm, lse_hbm,
               out_hbm, psum_hbm,
               idx_v, lidx_v, ltgt_v, fidx_v, tv_v, lsev_v,
               slab0, slab1, st0, st1, acc_v,
               ssem, lsem0, lsem1, osem0a, osem0b, osem1a, osem1b):
    wid = lax.axis_index("s") * 2 + lax.axis_index("c")
    pair = lax.div(wid, 2)
    half = lax.rem(wid, 2)
    i0 = pair * WCOL
    vt0 = half * VHALF
    nvt = jnp.where(half == 0, VHALF, VSLABS - VHALF)

    pltpu.sync_copy(idx_hbm.at[pl.ds(i0, WCOL)], idx_v)

    li0 = wid * PERW
    pltpu.sync_copy(idx_hbm.at[pl.ds(li0, PERW)], lidx_v)
    pltpu.sync_copy(tgt_hbm.at[pl.ds(li0, PERW)], ltgt_v)

    def flat_ids(g, _):
        sl = pl.ds(g * LANES, LANES)
        fidx_v[sl] = ltgt_v[sl] * VOCAB + lidx_v[sl]
        return 0
    lax.fori_loop(0, PERW // LANES, flat_ids, 0)

    def elem_gather_start(k, _):
        sl = pl.ds(k * SG, SG)
        pltpu.make_async_copy(ttf_hbm.at[fidx_v.at[sl]],
                              tv_v.at[sl], ssem).start()
        pltpu.make_async_copy(lse_hbm.at[lidx_v.at[sl]],
                              lsev_v.at[sl], ssem).start()
        return 0
    lax.fori_loop(0, PERW // SG, elem_gather_start, 0)

    def loss_tail():
        def elem_gather_wait(k, _):
            sl = pl.ds(k * SG, SG)
            pltpu.make_async_copy(ttf_hbm.at[fidx_v.at[sl]],
                                  tv_v.at[sl], ssem).wait()
            pltpu.make_async_copy(lse_hbm.at[lidx_v.at[sl]],
                                  lsev_v.at[sl], ssem).wait()
            return 0
        lax.fori_loop(0, PERW // SG, elem_gather_wait, 0)

        def loss_acc(g, acc):
            sl = pl.ds(g * LANES, LANES)
            return acc + (lsev_v[sl] - tv_v[sl])
        acc = lax.fori_loop(0, PERW // LANES, loss_acc,
                            jnp.zeros((LANES,), jnp.float32))
        acc_v[...] = acc
        pltpu.sync_copy(acc_v, psum_hbm.at[pl.ds(wid * LANES, LANES)])

    def slab_start(vt, slab, lsem):
        pltpu.make_async_copy(
            ttf_hbm.at[pl.ds(vt * (8 * CPAD), 8 * CPAD)], slab, lsem).start()

    def slab_wait(vt, slab, lsem):
        pltpu.make_async_copy(
            ttf_hbm.at[pl.ds(vt * (8 * CPAD), 8 * CPAD)], slab, lsem).wait()

    HA = 1536
    HB = WCOL - HA

    def half_start(vt, st, osem, off, hw):
        pltpu.make_async_copy(
            st.at[pl.ds(0, 8), pl.ds(off, hw)],
            out_hbm.at[pl.ds(vt * 8, 8), pl.ds(i0 + off, hw)], osem).start()

    def half_wait(st, osem, off, hw):
        pltpu.make_async_copy(
            st.at[pl.ds(0, 8), pl.ds(off, hw)],
            out_hbm.at[pl.ds(0, 8), pl.ds(i0 + off, hw)], osem).wait()

    def fill(slab_a, st, g_lo, g_hi):
        @plsc.parallel_loop(g_lo, g_hi, unroll=8)
        def _(g):
            idxg = idx_v[pl.ds(g * LANES, LANES)]
            for j in range(8):
                v = plsc.load_gather(slab_a, [idxg + (j * CPAD)])
                st[j, pl.ds(g * LANES, LANES)] = v

    def inner(vt, k, slab_a, lsem_a, slab_b, lsem_b, st, osem_a, osem_b):
        slab_wait(vt, slab_a, lsem_a)

        @pl.when(k + 1 < nvt)
        def _():
            slab_start(vt + 1, slab_b, lsem_b)

        @pl.when(k >= 2)
        def _():
            half_wait(st, osem_a, 0, HA)
        fill(slab_a, st, 0, HA // LANES)
        half_start(vt, st, osem_a, 0, HA)

        @pl.when(k >= 2)
        def _():
            half_wait(st, osem_b, HA, HB)
        fill(slab_a, st, HA // LANES, WCOL // LANES)
        half_start(vt, st, osem_b, HA, HB)

    slab_start(vt0, slab0, lsem0)

    def vt_body(k, _):
        vt = vt0 + k
        @pl.when(lax.rem(k, 2) == 0)
        def _():
            inner(vt, k, slab0, lsem0, slab1, lsem1, st0, osem0a, osem0b)
        @pl.when(lax.rem(k, 2) == 1)
        def _():
            inner(vt, k, slab1, lsem1, slab0, lsem0, st1, osem1a, osem1b)
        return 0

    lax.fori_loop(0, nvt, vt_body, 0)
    loss_tail()
    half_wait(st0, osem0a, 0, HA)
    half_wait(st0, osem0b, HA, HB)
    half_wait(st1, osem1a, 0, HA)
    half_wait(st1, osem1b, HA, HB)


def kernel(idx, targets, table):
    idx_flat = idx.reshape(-1).astype(jnp.int32)
    tgt_flat = targets.reshape(-1).astype(jnp.int32)
    ttf = table.T.reshape(-1)

    lse = pl.pallas_call(
        _lse_body,
        out_shape=jax.ShapeDtypeStruct((VOCAB, 1), jnp.float32),
    )(table)
    lse = lse.reshape(VOCAB)

    sc = functools.partial(
        pl.kernel,
        mesh=plsc.VectorSubcoreMesh(core_axis_name="c", subcore_axis_name="s"),
        out_type=[
            jax.ShapeDtypeStruct((VOCAB, N), jnp.float32),
            jax.ShapeDtypeStruct((NW * LANES,), jnp.float32),
        ],
        scratch_types=[
            pltpu.VMEM((WCOL,), jnp.int32),
            pltpu.VMEM((PERW,), jnp.int32),
            pltpu.VMEM((PERW,), jnp.int32),
            pltpu.VMEM((PERW,), jnp.int32),
            pltpu.VMEM((PERW,), jnp.float32),
            pltpu.VMEM((PERW,), jnp.float32),
            pltpu.VMEM((8 * CPAD,), jnp.float32),
            pltpu.VMEM((8 * CPAD,), jnp.float32),
            pltpu.VMEM((8, WCOL), jnp.float32),
            pltpu.VMEM((8, WCOL), jnp.float32),
            pltpu.VMEM((LANES,), jnp.float32),
            pltpu.SemaphoreType.DMA,
            pltpu.SemaphoreType.DMA,
            pltpu.SemaphoreType.DMA,
            pltpu.SemaphoreType.DMA,
            pltpu.SemaphoreType.DMA,
            pltpu.SemaphoreType.DMA,
            pltpu.SemaphoreType.DMA,
        ],
        compiler_params=pltpu.CompilerParams(
            use_tc_tiling_on_sc=True, needs_layout_passes=False,
            disable_bounds_checks=True),
    )(_sc_gather)
    outT, psums = sc(idx_flat, tgt_flat, ttf, lse)
    logits2 = outT.T

    fin = pl.pallas_call(
        _fin_body,
        out_shape=jax.ShapeDtypeStruct((1, 1), jnp.float32),
    )(psums)
    loss = fin[0, 0]
    return (logits2, loss)
